# baseline (speedup 1.0000x reference)
import jax
import jax.numpy as jnp
from jax.experimental import pallas as pl
from jax.experimental.pallas import tpu as pltpu

_C = 16
_N = 1048576
_W = 1024
_R = (_C * _N) // _W
_BR = 512
_BLKS_PER_C = (_N // _W) // _BR


def _body(loss_ref, succ_ref, out_ref, acc_ref):
    i = pl.program_id(0)

    @pl.when(i == 0)
    def _init():
        acc_ref[...] = jnp.zeros_like(acc_ref)

    c = i // _BLKS_PER_C
    x = loss_ref[...]
    masked = jnp.where(succ_ref[...] == 1, x, 0.0)
    pt = jnp.sum(masked, axis=0, keepdims=True)
    pa = jnp.sum(x, axis=0, keepdims=True)
    acc_ref[pl.ds(c, 1), :] += pt
    acc_ref[pl.ds(_C + c, 1), :] += pa

    @pl.when(i == pl.num_programs(0) - 1)
    def _fini():
        ts = jnp.sum(acc_ref[0:_C, :], axis=1, keepdims=True)
        tt = jnp.sum(acc_ref[_C:2 * _C, :], axis=1, keepdims=True)
        lv = jnp.log(ts / tt)
        kl = jnp.maximum(lv * lv - 0.01, 0.0)
        out_ref[...] = jnp.sum(kl, axis=0, keepdims=True)


def kernel(lossTensor, lcSuccesses):
    loss2 = lossTensor.reshape(_R, _W)
    succ2 = lcSuccesses.reshape(_R, _W)
    grid = _R // _BR
    out = pl.pallas_call(
        _body,
        grid=(grid,),
        in_specs=[
            pl.BlockSpec((_BR, _W), lambda i: (i, 0)),
            pl.BlockSpec((_BR, _W), lambda i: (i, 0)),
        ],
        out_specs=pl.BlockSpec((1, 1), lambda i: (0, 0)),
        out_shape=jax.ShapeDtypeStruct((1, 1), jnp.float32),
        scratch_shapes=[pltpu.VMEM((2 * _C, _W), jnp.float32)],
        compiler_params=pltpu.CompilerParams(
            dimension_semantics=("arbitrary",),
        ),
    )(loss2, succ2)
    return out[0, 0]
